# Initial kernel scaffold; baseline (speedup 1.0000x reference)
import jax
import jax.numpy as jnp
from jax.experimental import pallas as pl


def kernel(x, pos_table):
    raise NotImplementedError("write your pallas kernel here")


# speedup vs baseline: 12.1440x; 12.1440x over previous
import functools

import jax
import jax.numpy as jnp
from jax import lax
from jax.experimental import pallas as pl
from jax.experimental.pallas import tpu as pltpu
from jax.experimental.pallas import tpu_sc as plsc

_B = 4
_L = 8192
_NC = 2
_NS = 16
_NW = _NC * _NS
_CHUNK = _L // _NW
_LANES = 16

_mesh = plsc.VectorSubcoreMesh(core_axis_name="c", subcore_axis_name="s")


@functools.partial(
    pl.kernel,
    mesh=_mesh,
    out_type=jax.ShapeDtypeStruct((_B, _L), jnp.float32),
    scratch_types=[
        pltpu.VMEM((_B, _CHUNK), jnp.float32),
        pltpu.VMEM((_CHUNK,), jnp.float32),
    ],
)
def _pos_add_sc(x_hbm, pos_hbm, out_hbm, x_v, pos_v):
    wid = lax.axis_index("s") * _NC + lax.axis_index("c")
    base = wid * _CHUNK
    pltpu.sync_copy(pos_hbm.at[pl.ds(base, _CHUNK)], pos_v)
    pltpu.sync_copy(x_hbm.at[:, pl.ds(base, _CHUNK)], x_v)
    for b in range(_B):
        for i in range(_CHUNK // _LANES):
            sl = pl.ds(i * _LANES, _LANES)
            x_v[b, sl] = x_v[b, sl] + pos_v[sl]
    pltpu.sync_copy(x_v, out_hbm.at[:, pl.ds(base, _CHUNK)])


def kernel(x, pos_table):
    pos = pos_table.reshape(-1)[: x.shape[1]]
    return _pos_add_sc(x, pos)
